# baseline (speedup 1.0000x reference)
import jax
import jax.numpy as jnp
from jax.experimental import pallas as pl

PC_MIN = jnp.array([0.0, -40.0, -3.0], dtype=jnp.float32)
PS = 0.4
GX, GY = 176, 200
B, NP = 4, 100000
NSEG = B * GX * GY


def _noop(x_ref, o_ref):
    o_ref[...] = x_ref[...]


def kernel(points, xyz_batch_cnt, W1, b1, W2, b2):
    batch_ids = jnp.repeat(jnp.arange(B, dtype=jnp.int32), xyz_batch_cnt,
                           total_repeat_length=B * NP)
    xyz = points[:, :3] - PC_MIN
    ix = jnp.clip(jnp.floor(xyz[:, 0] / PS).astype(jnp.int32), 0, GX - 1)
    iy = jnp.clip(jnp.floor(xyz[:, 1] / PS).astype(jnp.int32), 0, GY - 1)
    pid = batch_ids * (GX * GY) + iy * GX + ix
    ones = jnp.ones((points.shape[0], 1), dtype=jnp.float32)
    cnt = jax.ops.segment_sum(ones, pid, num_segments=NSEG)
    ssum = jax.ops.segment_sum(xyz, pid, num_segments=NSEG)
    mean = ssum / jnp.maximum(cnt, 1.0)
    f_cluster = xyz - mean[pid]
    cx = (ix.astype(jnp.float32) + 0.5) * PS
    cy = (iy.astype(jnp.float32) + 0.5) * PS
    cz = jnp.full_like(cx, 2.0)
    f_center = jnp.stack([xyz[:, 0] - cx, xyz[:, 1] - cy, xyz[:, 2] - cz], axis=1)
    feats = jnp.concatenate([points, f_cluster, f_center, xyz], axis=1)
    h = jax.nn.relu(feats @ W1 + b1)
    h = jax.nn.relu(h @ W2 + b2)
    pooled = jax.ops.segment_max(h, pid, num_segments=NSEG)
    pooled = jnp.where(cnt > 0, pooled, 0.0)
    top = pl.pallas_call(
        _noop, out_shape=jax.ShapeDtypeStruct((8, 256), jnp.float32),
    )(pooled[:8])
    return jnp.concatenate([top, pooled[8:]], axis=0)


# speedup vs baseline: 1.0175x; 1.0175x over previous
import functools
import jax
import jax.numpy as jnp
from jax.experimental import pallas as pl

PC_MIN = jnp.array([0.0, -40.0, -3.0], dtype=jnp.float32)
PS = 0.4
GX, GY = 176, 200
B, NP = 4, 100000
N = B * NP
NSEG = B * GX * GY

BLK = 3200


def _mlp_body(ft_ref, w1_ref, b1_ref, w2_ref, b2_ref, o_ref):
    ft = ft_ref[...]
    h1 = jax.lax.dot_general(ft, w1_ref[...], (((0,), (0,)), ((), ())),
                             preferred_element_type=jnp.float32)
    h1 = jnp.maximum(h1 + b1_ref[...], 0.0)
    h2 = jax.lax.dot_general(h1, w2_ref[...], (((1,), (0,)), ((), ())),
                             preferred_element_type=jnp.float32)
    o_ref[...] = jnp.maximum(h2 + b2_ref[...], 0.0)


def _mlp(feats_t, w1p, b1, w2, b2):
    grid = (N // BLK,)
    return pl.pallas_call(
        _mlp_body,
        grid=grid,
        in_specs=[
            pl.BlockSpec((16, BLK), lambda i: (0, i)),
            pl.BlockSpec((16, 128), lambda i: (0, 0)),
            pl.BlockSpec((1, 128), lambda i: (0, 0)),
            pl.BlockSpec((128, 256), lambda i: (0, 0)),
            pl.BlockSpec((1, 256), lambda i: (0, 0)),
        ],
        out_specs=pl.BlockSpec((BLK, 256), lambda i: (i, 0)),
        out_shape=jax.ShapeDtypeStruct((N, 256), jnp.float32),
    )(feats_t, w1p, b1.reshape(1, 128), w2, b2.reshape(1, 256))


def kernel(points, xyz_batch_cnt, W1, b1, W2, b2):
    batch_ids = jnp.repeat(jnp.arange(B, dtype=jnp.int32), xyz_batch_cnt,
                           total_repeat_length=N)
    xyz = points[:, :3] - PC_MIN
    ix = jnp.clip(jnp.floor(xyz[:, 0] / PS).astype(jnp.int32), 0, GX - 1)
    iy = jnp.clip(jnp.floor(xyz[:, 1] / PS).astype(jnp.int32), 0, GY - 1)
    pid = batch_ids * (GX * GY) + iy * GX + ix
    ones = jnp.ones((N, 1), dtype=jnp.float32)
    cnt = jax.ops.segment_sum(ones, pid, num_segments=NSEG)
    ssum = jax.ops.segment_sum(xyz, pid, num_segments=NSEG)
    mean = ssum / jnp.maximum(cnt, 1.0)
    f_cluster = xyz - mean[pid]
    cx = (ix.astype(jnp.float32) + 0.5) * PS
    cy = (iy.astype(jnp.float32) + 0.5) * PS
    cz = jnp.full_like(cx, 2.0)
    f_center = jnp.stack([xyz[:, 0] - cx, xyz[:, 1] - cy, xyz[:, 2] - cz], axis=1)
    feats = jnp.concatenate([points, f_cluster, f_center, xyz,
                             jnp.zeros((N, 3), jnp.float32)], axis=1)
    w1p = jnp.concatenate([W1, jnp.zeros((3, 128), jnp.float32)], axis=0)
    h = _mlp(feats.T, w1p, b1, W2, b2)
    pooled = jax.ops.segment_max(h, pid, num_segments=NSEG)
    pooled = jnp.where(cnt > 0, pooled, 0.0)
    return pooled
